# baseline (speedup 1.0000x reference)
import functools

import jax
import jax.numpy as jnp
from jax import lax
from jax.experimental import pallas as pl
from jax.experimental.pallas import tpu as pltpu
from jax.experimental.pallas import tpu_sc as plsc

N = 10000
D = 128
E = 320000
NPAD = 10240
NC = 2
NS = 16
NW = NC * NS
K = 128
EPAD = 327680
EPT = EPAD // NW
CHUNKS = EPT // K
RPT = NPAD // NS
PADROW = 10200
DG = 8

_MESH = plsc.VectorSubcoreMesh(core_axis_name="c", subcore_axis_name="s")


def _deg_body(dst_hbm, out0, out1, dst_v, ones_v, zbuf, acc_sh):
    c = lax.axis_index("c")
    s = lax.axis_index("s")
    wid = s * NC + c
    z16 = jnp.zeros((16,), jnp.float32)
    one16 = jnp.ones((16,), jnp.float32)

    def zb(k, _):
        zbuf[k // 8, pl.ds((k % 8) * 16, 16)] = z16
        return 0

    lax.fori_loop(0, K * 8, zb, 0)

    def ob(k, _):
        ones_v[k // 8, pl.ds((k % 8) * 16, 16)] = one16
        return 0

    lax.fori_loop(0, K * 8, ob, 0)
    for b in range(RPT // 128):
        pltpu.sync_copy(zbuf, acc_sh.at[pl.ds(s * RPT + b * 128, 128)])
    plsc.subcore_barrier()

    base0 = wid * EPT

    def chunk(i, _):
        pltpu.sync_copy(dst_hbm.at[pl.ds(base0 + i * K, K)], dst_v)
        pltpu.sync_copy(ones_v, acc_sh.at[dst_v], add=True)
        return 0

    lax.fori_loop(0, CHUNKS, chunk, 0)
    plsc.subcore_barrier()

    @pl.when(c == 0)
    def _():
        pltpu.sync_copy(acc_sh.at[pl.ds(s * RPT, RPT)],
                        out0.at[pl.ds(s * RPT, RPT)])

    @pl.when(c == 1)
    def _():
        pltpu.sync_copy(acc_sh.at[pl.ds(s * RPT, RPT)],
                        out1.at[pl.ds(s * RPT, RPT)])


_deg_kernel = functools.partial(
    pl.kernel, _deg_body, mesh=_MESH,
    out_type=[jax.ShapeDtypeStruct((NPAD, 128), jnp.float32),
              jax.ShapeDtypeStruct((NPAD, 128), jnp.float32)],
    scratch_types=[
        pltpu.VMEM((K,), jnp.int32),
        pltpu.VMEM((K, 128), jnp.float32),
        pltpu.VMEM((128, 128), jnp.float32),
        pltpu.VMEM_SHARED((NPAD, 128), jnp.float32),
    ],
)()


def _scat_body(t_hbm, src_hbm, dst_hbm, out0, out1,
               src_v, dst_v, rows_v, zbuf, acc_sh, sem):
    c = lax.axis_index("c")
    s = lax.axis_index("s")
    wid = s * NC + c
    z16 = jnp.zeros((16,), jnp.float32)

    def zb(k, _):
        zbuf[k // 8, pl.ds((k % 8) * 16, 16)] = z16
        return 0

    lax.fori_loop(0, K * 8, zb, 0)
    for b in range(RPT // 128):
        pltpu.sync_copy(zbuf, acc_sh.at[pl.ds(s * RPT + b * 128, 128)])
    plsc.subcore_barrier()

    base0 = wid * EPT

    def chunk(i, _):
        base = base0 + i * K
        pltpu.sync_copy(src_hbm.at[pl.ds(base, K)], src_v)
        pltpu.sync_copy(dst_hbm.at[pl.ds(base, K)], dst_v)
        pltpu.async_copy(t_hbm.at[src_v], rows_v, sem).wait()
        pltpu.sync_copy(rows_v, acc_sh.at[dst_v], add=True)
        return 0

    lax.fori_loop(0, CHUNKS, chunk, 0)
    plsc.subcore_barrier()

    @pl.when(c == 0)
    def _():
        pltpu.sync_copy(acc_sh.at[pl.ds(s * RPT, RPT)],
                        out0.at[pl.ds(s * RPT, RPT)])

    @pl.when(c == 1)
    def _():
        pltpu.sync_copy(acc_sh.at[pl.ds(s * RPT, RPT)],
                        out1.at[pl.ds(s * RPT, RPT)])


_scat_kernel = functools.partial(
    pl.kernel, _scat_body, mesh=_MESH,
    out_type=[jax.ShapeDtypeStruct((NPAD, 128), jnp.float32),
              jax.ShapeDtypeStruct((NPAD, 128), jnp.float32)],
    scratch_types=[
        pltpu.VMEM((K,), jnp.int32),
        pltpu.VMEM((K,), jnp.int32),
        pltpu.VMEM((K, 128), jnp.float32),
        pltpu.VMEM((128, 128), jnp.float32),
        pltpu.VMEM_SHARED((NPAD, 128), jnp.float32),
        pltpu.SemaphoreType.DMA,
    ],
)()


_SPEC_FULL = pl.BlockSpec((NPAD, 128), lambda: (0, 0))
_SPEC_COL = pl.BlockSpec((NPAD, 1), lambda: (0, 0))
_SPEC_W = pl.BlockSpec((128, 128), lambda: (0, 0))
_SPEC_B = pl.BlockSpec((1, 128), lambda: (0, 0))


def _dinv(da_ref, db_ref):
    return lax.rsqrt(da_ref[...] + db_ref[...] + 1.0)


def _b1_body(x_ref, w_ref, da_ref, db_ref, o_ref):
    dinv = _dinv(da_ref, db_ref)
    o_ref[...] = jnp.dot(x_ref[...], w_ref[...],
                         preferred_element_type=jnp.float32) * dinv


def _b2_body(sa_ref, sb_ref, t_ref, da_ref, db_ref, b_ref, w_ref, o_ref):
    dinv = _dinv(da_ref, db_ref)
    agg = (sa_ref[...] + sb_ref[...] + t_ref[...]) * dinv + b_ref[...]
    h = jnp.maximum(agg, 0.0)
    t2 = jnp.dot(h, w_ref[...], preferred_element_type=jnp.float32) * dinv
    rows = lax.broadcasted_iota(jnp.int32, (NPAD, 128), 0)
    o_ref[...] = jnp.where(rows < N, t2, 0.0)


def _b3_body(sa_ref, sb_ref, t_ref, da_ref, db_ref, b_ref, o_ref):
    dinv = _dinv(da_ref, db_ref)
    o_ref[...] = (sa_ref[...] + sb_ref[...] + t_ref[...]) * dinv + b_ref[...]


def kernel(x, edge_index, W1, b1, W2, b2):
    src = edge_index[0].astype(jnp.int32)
    dst = edge_index[1].astype(jnp.int32)
    pad = jnp.full((EPAD - E,), PADROW, jnp.int32)
    src_p = jnp.concatenate([src, pad])
    dst_p = jnp.concatenate([dst, pad])
    x_pad = jnp.pad(x, ((0, NPAD - N), (0, 0)))
    b1r = b1.reshape(1, 128)
    b2r = b2.reshape(1, 128)

    d0, d1 = _deg_kernel(dst_p)
    d0c = d0[:, :1]
    d1c = d1[:, :1]

    f32 = jnp.float32
    t1 = pl.pallas_call(
        _b1_body, out_shape=jax.ShapeDtypeStruct((NPAD, 128), f32),
        in_specs=[_SPEC_FULL, _SPEC_W, _SPEC_COL, _SPEC_COL],
        out_specs=_SPEC_FULL,
    )(x_pad, W1, d0c, d1c)

    s1a, s1b = _scat_kernel(t1, src_p, dst_p)

    t2 = pl.pallas_call(
        _b2_body, out_shape=jax.ShapeDtypeStruct((NPAD, 128), f32),
        in_specs=[_SPEC_FULL, _SPEC_FULL, _SPEC_FULL, _SPEC_COL, _SPEC_COL,
                  _SPEC_B, _SPEC_W],
        out_specs=_SPEC_FULL,
    )(s1a, s1b, t1, d0c, d1c, b1r, W2)

    s2a, s2b = _scat_kernel(t2, src_p, dst_p)

    z = pl.pallas_call(
        _b3_body, out_shape=jax.ShapeDtypeStruct((NPAD, 128), f32),
        in_specs=[_SPEC_FULL, _SPEC_FULL, _SPEC_FULL, _SPEC_COL, _SPEC_COL,
                  _SPEC_B],
        out_specs=_SPEC_FULL,
    )(s2a, s2b, t2, d0c, d1c, b2r)

    return z[:N]


# speedup vs baseline: 1.5749x; 1.5749x over previous
import functools

import jax
import jax.numpy as jnp
from jax import lax
from jax.experimental import pallas as pl
from jax.experimental.pallas import tpu as pltpu
from jax.experimental.pallas import tpu_sc as plsc

N = 10000
D = 128
E = 320000
NPAD = 10240
NC = 2
NS = 16
NW = NC * NS
K = 128
EPAD = 323584
EPT = EPAD // NW
CHUNKS = EPT // K
RPT = NPAD // NS
PADROW = 10200
DG = 8

_MESH = plsc.VectorSubcoreMesh(core_axis_name="c", subcore_axis_name="s")


def _deg_body(dst_hbm, out0, out1, dst_v, ones_v, zbuf, acc_sh):
    c = lax.axis_index("c")
    s = lax.axis_index("s")
    wid = s * NC + c
    z16 = jnp.zeros((16,), jnp.float32)
    one16 = jnp.ones((16,), jnp.float32)

    def zb(k, _):
        zbuf[k // 8, pl.ds((k % 8) * 16, 16)] = z16
        return 0

    lax.fori_loop(0, K * 8, zb, 0)

    def ob(k, _):
        ones_v[k // 8, pl.ds((k % 8) * 16, 16)] = one16
        return 0

    lax.fori_loop(0, K * 8, ob, 0)
    for b in range(RPT // 128):
        pltpu.sync_copy(zbuf, acc_sh.at[pl.ds(s * RPT + b * 128, 128)])
    plsc.subcore_barrier()

    base0 = wid * EPT

    def chunk(i, _):
        pltpu.sync_copy(dst_hbm.at[pl.ds(base0 + i * K, K)], dst_v)
        pltpu.sync_copy(ones_v, acc_sh.at[dst_v], add=True)
        return 0

    lax.fori_loop(0, CHUNKS, chunk, 0)
    plsc.subcore_barrier()

    @pl.when(c == 0)
    def _():
        pltpu.sync_copy(acc_sh.at[pl.ds(s * RPT, RPT)],
                        out0.at[pl.ds(s * RPT, RPT)])

    @pl.when(c == 1)
    def _():
        pltpu.sync_copy(acc_sh.at[pl.ds(s * RPT, RPT)],
                        out1.at[pl.ds(s * RPT, RPT)])


_deg_kernel = functools.partial(
    pl.kernel, _deg_body, mesh=_MESH,
    out_type=[jax.ShapeDtypeStruct((NPAD, 128), jnp.float32),
              jax.ShapeDtypeStruct((NPAD, 128), jnp.float32)],
    scratch_types=[
        pltpu.VMEM((K,), jnp.int32),
        pltpu.VMEM((K, 128), jnp.float32),
        pltpu.VMEM((128, 128), jnp.float32),
        pltpu.VMEM_SHARED((NPAD, 128), jnp.float32),
    ],
)()


def _scat_body(t_hbm, src_hbm, dst_hbm, out0, out1,
               src_v, dst_v, rows_v, zbuf, acc_sh, sem):
    c = lax.axis_index("c")
    s = lax.axis_index("s")
    wid = s * NC + c
    z16 = jnp.zeros((16,), jnp.float32)

    def zb(k, _):
        zbuf[k // 8, pl.ds((k % 8) * 16, 16)] = z16
        return 0

    lax.fori_loop(0, K * 8, zb, 0)
    for b in range(RPT // 128):
        pltpu.sync_copy(zbuf, acc_sh.at[pl.ds(s * RPT + b * 128, 128)])
    plsc.subcore_barrier()

    base0 = wid * EPT

    def chunk(i, _):
        base = base0 + i * K
        pltpu.sync_copy(src_hbm.at[pl.ds(base, K)], src_v)
        pltpu.sync_copy(dst_hbm.at[pl.ds(base, K)], dst_v)
        pltpu.async_copy(t_hbm.at[src_v], rows_v, sem).wait()
        pltpu.sync_copy(rows_v, acc_sh.at[dst_v], add=True)
        return 0

    lax.fori_loop(0, CHUNKS, chunk, 0)
    plsc.subcore_barrier()

    @pl.when(c == 0)
    def _():
        pltpu.sync_copy(acc_sh.at[pl.ds(s * RPT, RPT)],
                        out0.at[pl.ds(s * RPT, RPT)])

    @pl.when(c == 1)
    def _():
        pltpu.sync_copy(acc_sh.at[pl.ds(s * RPT, RPT)],
                        out1.at[pl.ds(s * RPT, RPT)])


_scat_kernel = functools.partial(
    pl.kernel, _scat_body, mesh=_MESH,
    out_type=[jax.ShapeDtypeStruct((NPAD, 128), jnp.float32),
              jax.ShapeDtypeStruct((NPAD, 128), jnp.float32)],
    scratch_types=[
        pltpu.VMEM((K,), jnp.int32),
        pltpu.VMEM((K,), jnp.int32),
        pltpu.VMEM((K, 128), jnp.float32),
        pltpu.VMEM((128, 128), jnp.float32),
        pltpu.VMEM_SHARED((NPAD, 128), jnp.float32),
        pltpu.SemaphoreType.DMA,
    ],
)()


_SPEC_FULL = pl.BlockSpec((NPAD, 128), lambda: (0, 0))
_SPEC_COL = pl.BlockSpec((NPAD, 1), lambda: (0, 0))
_SPEC_W = pl.BlockSpec((128, 128), lambda: (0, 0))
_SPEC_B = pl.BlockSpec((1, 128), lambda: (0, 0))


def _dinv(da_ref, db_ref):
    return lax.rsqrt(da_ref[...] + db_ref[...] + 1.0)


def _b1_body(x_ref, w_ref, da_ref, db_ref, o_ref):
    dinv = _dinv(da_ref, db_ref)
    o_ref[...] = jnp.dot(x_ref[...], w_ref[...],
                         preferred_element_type=jnp.float32) * dinv


def _b2_body(sa_ref, sb_ref, t_ref, da_ref, db_ref, b_ref, w_ref, o_ref):
    dinv = _dinv(da_ref, db_ref)
    agg = (sa_ref[...] + sb_ref[...] + t_ref[...]) * dinv + b_ref[...]
    h = jnp.maximum(agg, 0.0)
    t2 = jnp.dot(h, w_ref[...], preferred_element_type=jnp.float32) * dinv
    rows = lax.broadcasted_iota(jnp.int32, (NPAD, 128), 0)
    o_ref[...] = jnp.where(rows < N, t2, 0.0)


def _b3_body(sa_ref, sb_ref, t_ref, da_ref, db_ref, b_ref, o_ref):
    dinv = _dinv(da_ref, db_ref)
    o_ref[...] = (sa_ref[...] + sb_ref[...] + t_ref[...]) * dinv + b_ref[...]


def kernel(x, edge_index, W1, b1, W2, b2):
    src = edge_index[0].astype(jnp.int32)
    dst = edge_index[1].astype(jnp.int32)
    pad = jnp.full((EPAD - E,), PADROW, jnp.int32)
    src_p = jnp.concatenate([src, pad])
    dst_p = jnp.concatenate([dst, pad])
    x_pad = jnp.pad(x, ((0, NPAD - N), (0, 0)))
    b1r = b1.reshape(1, 128)
    b2r = b2.reshape(1, 128)

    d0, d1 = _deg_kernel(dst_p)
    d0c = d0[:, :1]
    d1c = d1[:, :1]

    f32 = jnp.float32
    t1 = pl.pallas_call(
        _b1_body, out_shape=jax.ShapeDtypeStruct((NPAD, 128), f32),
        in_specs=[_SPEC_FULL, _SPEC_W, _SPEC_COL, _SPEC_COL],
        out_specs=_SPEC_FULL,
    )(x_pad, W1, d0c, d1c)

    s1a, s1b = _scat_kernel(t1, src_p, dst_p)

    t2 = pl.pallas_call(
        _b2_body, out_shape=jax.ShapeDtypeStruct((NPAD, 128), f32),
        in_specs=[_SPEC_FULL, _SPEC_FULL, _SPEC_FULL, _SPEC_COL, _SPEC_COL,
                  _SPEC_B, _SPEC_W],
        out_specs=_SPEC_FULL,
    )(s1a, s1b, t1, d0c, d1c, b1r, W2)

    s2a, s2b = _scat_kernel(t2, src_p, dst_p)

    z = pl.pallas_call(
        _b3_body, out_shape=jax.ShapeDtypeStruct((NPAD, 128), f32),
        in_specs=[_SPEC_FULL, _SPEC_FULL, _SPEC_FULL, _SPEC_COL, _SPEC_COL,
                  _SPEC_B],
        out_specs=_SPEC_FULL,
    )(s2a, s2b, t2, d0c, d1c, b2r)

    return z[:N]
